# baseline (speedup 1.0000x reference)
import functools

import jax
import jax.numpy as jnp
from jax import lax
from jax.experimental import pallas as pl
from jax.experimental.pallas import tpu as pltpu
from jax.experimental.pallas import tpu_sc as plsc

NUM_EMBED = 8
NUM_EMBEDDING = 256
EMBED_DIM = 64

NC = 2
NS = 16
NW = NC * NS

LANES = 16
CHUNK = 128
NBUF = 4


def _build(batch):
    total_rows = batch * NUM_EMBED
    rows_per_w = total_rows // NW
    n_per_w = batch // NW
    n_chunks = rows_per_w // CHUNK
    tab_rows = NUM_EMBED * NUM_EMBEDDING
    stage_rows = tab_rows // NS

    mesh = plsc.VectorSubcoreMesh(
        core_axis_name="c", subcore_axis_name="s", num_cores=NC,
        num_subcores=NS)

    @functools.partial(
        pl.kernel,
        out_type=jax.ShapeDtypeStruct((total_rows, EMBED_DIM), jnp.float32),
        mesh=mesh,
        compiler_params=pltpu.CompilerParams(
            needs_layout_passes=False, use_tc_tiling_on_sc=False),
        scratch_types=[
            pltpu.VMEM((n_per_w,), jnp.int32),
            pltpu.VMEM((n_chunks, CHUNK), jnp.int32),
            pltpu.VMEM((NBUF, CHUNK, EMBED_DIM), jnp.float32),
            pltpu.VMEM_SHARED((tab_rows, EMBED_DIM), jnp.float32),
            [pltpu.SemaphoreType.DMA] * NBUF,
            [pltpu.SemaphoreType.DMA] * NBUF,
        ],
    )
    def k(x_hbm, tab_hbm, out_hbm, x_v, idx_v, rows_v, tab_sp, gsems, wsems):
        sid = lax.axis_index("s")
        wid = sid * NC + lax.axis_index("c")
        nbase = wid * n_per_w
        rbase = wid * rows_per_w

        pltpu.sync_copy(
            tab_hbm.at[pl.ds(sid * stage_rows, stage_rows)],
            rows_v.at[0])
        pltpu.sync_copy(
            rows_v.at[0],
            tab_sp.at[pl.ds(sid * stage_rows, stage_rows)])

        pltpu.sync_copy(x_hbm.at[pl.ds(nbase, n_per_w)], x_v)

        lane = lax.iota(jnp.int32, 16)
        nsel = lax.shift_right_logical(lane, 1)
        pshift = lax.shift_left(lane & 1, 3)
        pbase = lax.shift_left(lane & 1, 8)

        def compute(j, _):
            for i in range(8):
                b, cb = i >> 2, i & 3
                x = plsc.load_gather(x_v, [nsel + (j * 16 + b * 8)])
                byte = lax.shift_right_logical(
                    jnp.abs(x), pshift + 16 * (cb & 1)) & 255
                idx_v[j, pl.ds(i * LANES, LANES)] = (
                    byte + (pbase + 512 * cb))
            return 0

        lax.fori_loop(0, n_chunks, compute, 0)

        plsc.subcore_barrier()

        def gather_start(j):
            b = j % NBUF
            return pltpu.async_copy(
                tab_sp.at[idx_v.at[j]],
                rows_v.at[b], gsems[b])

        def write_start(j):
            b = j % NBUF
            return pltpu.async_copy(
                rows_v.at[b],
                out_hbm.at[pl.ds(rbase + j * CHUNK, CHUNK)], wsems[b])

        gcp = [None] * n_chunks
        wcp = [None] * n_chunks
        for j in range(min(2, n_chunks)):
            gcp[j] = gather_start(j)
        for j in range(n_chunks):
            if j >= 2:
                wcp[j - 2].wait()
            if j + 2 < n_chunks:
                gcp[j + 2] = gather_start(j + 2)
            gcp[j].wait()
            wcp[j] = write_start(j)
        for j in range(max(0, n_chunks - 2), n_chunks):
            wcp[j].wait()

    return k


@jax.jit
def kernel(X, tables):
    batch = X.shape[0]
    tab2d = tables.reshape(NUM_EMBED * NUM_EMBEDDING, EMBED_DIM)
    out = _build(batch)(X, tab2d)
    out5 = out.reshape(batch // 8, 4, 8, 2, EMBED_DIM)
    return out5.transpose(0, 2, 1, 3, 4).reshape(
        batch, NUM_EMBED * EMBED_DIM)


# speedup vs baseline: 4.3338x; 4.3338x over previous
import functools

import jax
import jax.numpy as jnp
from jax import lax
from jax.experimental import pallas as pl
from jax.experimental.pallas import tpu as pltpu
from jax.experimental.pallas import tpu_sc as plsc

NUM_EMBED = 8
NUM_EMBEDDING = 256
EMBED_DIM = 64

NC = 2
NS = 16
NW = NC * NS

LANES = 16
CHUNK = 128
NBUF = 4


def _build(batch):
    total_rows = batch * NUM_EMBED
    rows_per_w = total_rows // NW
    n_per_w = batch // NW
    n_chunks = rows_per_w // CHUNK
    tab_rows = NUM_EMBED * NUM_EMBEDDING
    stage_rows = tab_rows // NS

    mesh = plsc.VectorSubcoreMesh(
        core_axis_name="c", subcore_axis_name="s", num_cores=NC,
        num_subcores=NS)

    @functools.partial(
        pl.kernel,
        out_type=jax.ShapeDtypeStruct((total_rows, EMBED_DIM), jnp.float32),
        mesh=mesh,
        compiler_params=pltpu.CompilerParams(
            needs_layout_passes=False, use_tc_tiling_on_sc=False),
        scratch_types=[
            pltpu.VMEM((n_per_w,), jnp.int32),
            pltpu.VMEM((n_chunks, CHUNK), jnp.int32),
            pltpu.VMEM((NBUF, CHUNK, EMBED_DIM), jnp.float32),
            pltpu.VMEM_SHARED((tab_rows, EMBED_DIM), jnp.float32),
            [pltpu.SemaphoreType.DMA] * NBUF,
            [pltpu.SemaphoreType.DMA] * NBUF,
        ],
    )
    def k(x_hbm, tab_hbm, out_hbm, x_v, idx_v, rows_v, tab_sp, gsems, wsems):
        sid = lax.axis_index("s")
        wid = sid * NC + lax.axis_index("c")
        nbase = wid * n_per_w
        rbase = wid * rows_per_w

        pltpu.sync_copy(
            tab_hbm.at[pl.ds(sid * stage_rows, stage_rows)], rows_v.at[0])
        pltpu.sync_copy(
            rows_v.at[0], tab_sp.at[pl.ds(sid * stage_rows, stage_rows)])

        pltpu.sync_copy(x_hbm.at[pl.ds(nbase, n_per_w)], x_v)

        lane = lax.iota(jnp.int32, 16)
        nsel = lax.shift_right_logical(lane, 3)
        shiftv = lax.shift_left(lane & 3, 3)
        basev = lax.shift_left(lane & 7, 8)

        def compute(j, _):
            for c in range(8):
                i = j * 8 + c
                x = plsc.load_gather(x_v, [nsel + 2 * i])
                byte = lax.shift_right_logical(jnp.abs(x), shiftv) & 255
                idx_v[j, pl.ds(c * LANES, LANES)] = basev + byte
            return 0

        lax.fori_loop(0, n_chunks, compute, 0)

        plsc.subcore_barrier()

        def gather_start(j):
            b = j % NBUF
            return pltpu.async_copy(
                tab_sp.at[idx_v.at[j]], rows_v.at[b], gsems[b])

        def write_start(j):
            b = j % NBUF
            return pltpu.async_copy(
                rows_v.at[b],
                out_hbm.at[pl.ds(rbase + j * CHUNK, CHUNK)], wsems[b])

        gcp = [None] * n_chunks
        wcp = [None] * n_chunks
        for j in range(min(2, n_chunks)):
            gcp[j] = gather_start(j)
        for j in range(n_chunks):
            if j >= 2:
                wcp[j - 2].wait()
            if j + 2 < n_chunks:
                gcp[j + 2] = gather_start(j + 2)
            gcp[j].wait()
            wcp[j] = write_start(j)
        for j in range(max(0, n_chunks - 2), n_chunks):
            wcp[j].wait()

    return k


NSPLIT = 2


@jax.jit
def kernel(X, tables):
    batch = X.shape[0]
    tab2d = tables.reshape(NUM_EMBED * NUM_EMBEDDING, EMBED_DIM)
    half = batch // NSPLIT
    k = _build(half)
    parts = [
        k(lax.slice_in_dim(X, s * half, (s + 1) * half), tab2d)
        .reshape(half, NUM_EMBED * EMBED_DIM)
        for s in range(NSPLIT)
    ]
    return jnp.concatenate(parts, axis=0)


# speedup vs baseline: 5.5630x; 1.2836x over previous
import functools

import jax
import jax.numpy as jnp
from jax import lax
from jax.experimental import pallas as pl
from jax.experimental.pallas import tpu as pltpu
from jax.experimental.pallas import tpu_sc as plsc

NUM_EMBED = 8
NUM_EMBEDDING = 256
EMBED_DIM = 64

NC = 2
NS = 16
NW = NC * NS

LANES = 16
CHUNK = 128
NBUF = 4


def _build(batch):
    total_rows = batch * NUM_EMBED
    rows_per_w = total_rows // NW
    n_per_w = batch // NW
    n_chunks = rows_per_w // CHUNK
    tab_rows = NUM_EMBED * NUM_EMBEDDING
    stage_rows = tab_rows // NS

    mesh = plsc.VectorSubcoreMesh(
        core_axis_name="c", subcore_axis_name="s", num_cores=NC,
        num_subcores=NS)

    @functools.partial(
        pl.kernel,
        out_type=jax.ShapeDtypeStruct((total_rows, EMBED_DIM), jnp.float32),
        mesh=mesh,
        compiler_params=pltpu.CompilerParams(
            needs_layout_passes=False, use_tc_tiling_on_sc=False),
        scratch_types=[
            pltpu.VMEM((n_per_w,), jnp.int32),
            pltpu.VMEM((n_chunks, CHUNK), jnp.int32),
            pltpu.VMEM((NBUF, CHUNK, EMBED_DIM), jnp.float32),
            pltpu.VMEM_SHARED((tab_rows, EMBED_DIM), jnp.float32),
            [pltpu.SemaphoreType.DMA] * NBUF,
            [pltpu.SemaphoreType.DMA] * NBUF,
        ],
    )
    def k(x_hbm, tab_hbm, out_hbm, x_v, idx_v, rows_v, tab_sp, gsems, wsems):
        sid = lax.axis_index("s")
        wid = sid * NC + lax.axis_index("c")
        nbase = wid * n_per_w
        rbase = wid * rows_per_w

        pltpu.sync_copy(x_hbm.at[pl.ds(nbase, n_per_w)], x_v)
        stage_cp = pltpu.async_copy(
            tab_hbm.at[pl.ds(sid * stage_rows, stage_rows)], rows_v.at[0],
            gsems[0])

        lane = lax.iota(jnp.int32, 16)
        nsel = lax.shift_right_logical(lane, 3)
        shiftv = lax.shift_left(lane & 3, 3)
        basev = lax.shift_left(lane & 7, 8)

        def compute(j, _):
            for c in range(8):
                i = j * 8 + c
                x = plsc.load_gather(x_v, [nsel + 2 * i])
                byte = lax.shift_right_logical(jnp.abs(x), shiftv) & 255
                idx_v[j, pl.ds(c * LANES, LANES)] = basev + byte
            return 0

        lax.fori_loop(0, n_chunks, compute, 0)

        stage_cp.wait()
        pltpu.sync_copy(
            rows_v.at[0], tab_sp.at[pl.ds(sid * stage_rows, stage_rows)])
        plsc.subcore_barrier()

        def gather_start(j):
            b = j % NBUF
            return pltpu.async_copy(
                tab_sp.at[idx_v.at[j]], rows_v.at[b], gsems[b])

        def write_start(j):
            b = j % NBUF
            return pltpu.async_copy(
                rows_v.at[b],
                out_hbm.at[pl.ds(rbase + j * CHUNK, CHUNK)], wsems[b])

        gcp = [None] * n_chunks
        wcp = [None] * n_chunks
        for j in range(min(2, n_chunks)):
            gcp[j] = gather_start(j)
        for j in range(n_chunks):
            if j >= 2:
                wcp[j - 2].wait()
            if j + 2 < n_chunks:
                gcp[j + 2] = gather_start(j + 2)
            gcp[j].wait()
            wcp[j] = write_start(j)
        for j in range(max(0, n_chunks - 2), n_chunks):
            wcp[j].wait()

    return k


@jax.jit
def kernel(X, tables):
    batch = X.shape[0]
    tab2d = tables.reshape(NUM_EMBED * NUM_EMBEDDING, EMBED_DIM)
    out = _build(batch)(X, tab2d)
    return out.reshape(batch, NUM_EMBED * EMBED_DIM)


# speedup vs baseline: 5.5813x; 1.0033x over previous
import functools

import jax
import jax.numpy as jnp
from jax import lax
from jax.experimental import pallas as pl
from jax.experimental.pallas import tpu as pltpu
from jax.experimental.pallas import tpu_sc as plsc

NUM_EMBED = 8
NUM_EMBEDDING = 256
EMBED_DIM = 64

NC = 2
NS = 16
NW = NC * NS

LANES = 16
CHUNK = 128
NBUF = 8


def _build(batch):
    total_rows = batch * NUM_EMBED
    rows_per_w = total_rows // NW
    n_per_w = batch // NW
    n_chunks = rows_per_w // CHUNK
    tab_rows = NUM_EMBED * NUM_EMBEDDING
    stage_rows = tab_rows // NS

    mesh = plsc.VectorSubcoreMesh(
        core_axis_name="c", subcore_axis_name="s", num_cores=NC,
        num_subcores=NS)

    @functools.partial(
        pl.kernel,
        out_type=jax.ShapeDtypeStruct((total_rows, EMBED_DIM), jnp.float32),
        mesh=mesh,
        compiler_params=pltpu.CompilerParams(
            needs_layout_passes=False, use_tc_tiling_on_sc=False),
        scratch_types=[
            pltpu.VMEM((n_per_w,), jnp.int32),
            pltpu.VMEM((n_chunks, CHUNK), jnp.int32),
            pltpu.VMEM((NBUF, CHUNK, EMBED_DIM), jnp.float32),
            pltpu.VMEM_SHARED((tab_rows, EMBED_DIM), jnp.float32),
            [pltpu.SemaphoreType.DMA] * NBUF,
            [pltpu.SemaphoreType.DMA] * NBUF,
        ],
    )
    def k(x_hbm, tab_hbm, out_hbm, x_v, idx_v, rows_v, tab_sp, gsems, wsems):
        sid = lax.axis_index("s")
        wid = sid * NC + lax.axis_index("c")
        nbase = wid * n_per_w
        rbase = wid * rows_per_w

        pltpu.sync_copy(x_hbm.at[pl.ds(nbase, n_per_w)], x_v)
        stage_cp = pltpu.async_copy(
            tab_hbm.at[pl.ds(sid * stage_rows, stage_rows)], rows_v.at[0],
            gsems[0])

        lane = lax.iota(jnp.int32, 16)
        nsel = lax.shift_right_logical(lane, 3)
        shiftv = lax.shift_left(lane & 3, 3)
        basev = lax.shift_left(lane & 7, 8)

        def compute(j, _):
            for c in range(8):
                i = j * 8 + c
                x = plsc.load_gather(x_v, [nsel + 2 * i])
                byte = lax.shift_right_logical(jnp.abs(x), shiftv) & 255
                idx_v[j, pl.ds(c * LANES, LANES)] = basev + byte
            return 0

        lax.fori_loop(0, n_chunks, compute, 0)

        stage_cp.wait()
        pltpu.sync_copy(
            rows_v.at[0], tab_sp.at[pl.ds(sid * stage_rows, stage_rows)])
        plsc.subcore_barrier()

        def gather_start(j):
            b = j % NBUF
            return pltpu.async_copy(
                tab_sp.at[idx_v.at[j]], rows_v.at[b], gsems[b])

        def write_start(j):
            b = j % NBUF
            return pltpu.async_copy(
                rows_v.at[b],
                out_hbm.at[pl.ds(rbase + j * CHUNK, CHUNK)], wsems[b])

        gcp = [None] * n_chunks
        wcp = [None] * n_chunks
        depth = 4
        for j in range(min(depth, n_chunks)):
            gcp[j] = gather_start(j)
        for j in range(n_chunks):
            if j >= depth:
                wcp[j - depth].wait()
            if j + depth < n_chunks:
                gcp[j + depth] = gather_start(j + depth)
            gcp[j].wait()
            wcp[j] = write_start(j)
        for j in range(max(0, n_chunks - depth), n_chunks):
            wcp[j].wait()

    return k


@jax.jit
def kernel(X, tables):
    batch = X.shape[0]
    tab2d = tables.reshape(NUM_EMBED * NUM_EMBEDDING, EMBED_DIM)
    out = _build(batch)(X, tab2d)
    return out.reshape(batch, NUM_EMBED * EMBED_DIM)
